# baseline (speedup 1.0000x reference)
import functools

import jax
import jax.numpy as jnp
from jax import lax
from jax.experimental import pallas as pl
from jax.experimental.pallas import tpu as pltpu
from jax.experimental.pallas import tpu_sc as plsc

N = 10000
D = 128
NC = 2
NS = 16
NW = NC * NS
NPAD = 10240
ROWS_PER_TILE = NPAD // NS
CHUNK = 128
NBLK = 10
BLK = N // NBLK


def _sc_edge_body(H, E_real, h_hbm, alsad_hbm, gmax_hbm, src_hbm, dst_hbm,
                  znum_hbm, zden_hbm, wnum_out, wden_out, src_v, dst_v, rows_v,
                  aidx_v, adidx_v, didx_v, as_vals, ad_vals, svals_v, gmax_v,
                  sem, sem2, wnum_s, wden_s):
    C = D // H
    cid = lax.axis_index("c")
    sid = lax.axis_index("s")
    wid = sid * NC + cid
    EPAD = src_hbm.shape[0]
    per_tile = EPAD // NW
    n_chunks = per_tile // CHUNK

    pltpu.sync_copy(gmax_hbm, gmax_v)

    pltpu.sync_copy(znum_hbm, wnum_s.at[pl.ds(sid * ROWS_PER_TILE, ROWS_PER_TILE)])
    pltpu.sync_copy(zden_hbm, wden_s.at[pl.ds(sid * ROWS_PER_TILE * 8,
                                              ROWS_PER_TILE * 8)])
    plsc.subcore_barrier()

    lanes = lax.broadcasted_iota(jnp.int32, (16,), 0)

    def chunk_body(ch, carry):
        base = wid * per_tile + ch * CHUNK
        di = pltpu.async_copy(src_hbm.at[pl.ds(base, CHUNK)], src_v, sem)
        dj = pltpu.async_copy(dst_hbm.at[pl.ds(base, CHUNK)], dst_v, sem)
        di.wait()
        dj.wait()
        descs = [pltpu.async_copy(h_hbm.at[src_v], rows_v, sem)]
        for j in range(CHUNK // 16):
            jsl = pl.ds(j * 16, 16)
            sv = src_v[jsl]
            dv = dst_v[jsl]
            for hd in range(H):
                aidx_v[hd, jsl] = sv * 8 + hd
                adidx_v[hd, jsl] = dv * 8 + (H + hd)
                didx_v[hd, jsl] = dv * 8 + hd
        for hd in range(H):
            descs.append(pltpu.async_copy(
                alsad_hbm.at[aidx_v.at[hd]], as_vals.at[hd], sem2))
            descs.append(pltpu.async_copy(
                alsad_hbm.at[adidx_v.at[hd]], ad_vals.at[hd], sem2))
        for de in descs:
            de.wait()
        for j in range(CHUNK // 16):
            j16 = j * 16 + lanes
            jsl = pl.ds(j * 16, 16)
            valid = (base + j16) < E_real
            for hd in range(H):
                ts = as_vals[hd, jsl]
                td = ad_vals[hd, jsl]
                a = ts + td
                lr = jnp.maximum(a, 0.2 * a)
                gs = gmax_v[hd, pl.ds(16, 16)]
                gd = td + gs
                M = jnp.maximum(gd, 0.2 * gd)
                s = jnp.where(valid, jnp.exp(lr - M), 0.0)
                svals_v[hd, jsl] = s
                for c in range(hd * C, (hd + 1) * C):
                    cc = jnp.full((16,), c, jnp.int32)
                    colv = plsc.load_gather(rows_v, [j16, cc])
                    plsc.store_scatter(rows_v, [j16, cc], colv * s)
        outs = [pltpu.async_copy(rows_v, wnum_s.at[dst_v], sem, add=True)]
        for hd in range(H):
            outs.append(pltpu.async_copy(
                svals_v.at[hd], wden_s.at[didx_v.at[hd]], sem2, add=True))
        for de in outs:
            de.wait()
        return carry

    lax.fori_loop(0, n_chunks, chunk_body, 0)
    plsc.subcore_barrier()

    rsl = pl.ds(sid * ROWS_PER_TILE, ROWS_PER_TILE)
    fsl = pl.ds(sid * ROWS_PER_TILE * 8, ROWS_PER_TILE * 8)
    pltpu.sync_copy(wnum_s.at[rsl], wnum_out.at[cid, rsl])
    pltpu.sync_copy(wden_s.at[fsl], wden_out.at[cid, fsl])


def _sc_edge_call(H, h, alsad, gmax, src_pad, dst_pad, e_real):
    mesh = plsc.VectorSubcoreMesh(core_axis_name="c", subcore_axis_name="s")
    znum = jnp.zeros((ROWS_PER_TILE, D), jnp.float32)
    zden = jnp.zeros((ROWS_PER_TILE * 8,), jnp.float32)
    kern = pl.kernel(
        functools.partial(_sc_edge_body, H, e_real),
        out_type=(
            jax.ShapeDtypeStruct((NC, NPAD, D), jnp.float32),
            jax.ShapeDtypeStruct((NC, NPAD * 8), jnp.float32),
        ),
        mesh=mesh,
        scratch_types=[
            pltpu.VMEM((CHUNK,), jnp.int32),
            pltpu.VMEM((CHUNK,), jnp.int32),
            pltpu.VMEM((CHUNK, D), jnp.float32),
            pltpu.VMEM((8, CHUNK), jnp.int32),
            pltpu.VMEM((8, CHUNK), jnp.int32),
            pltpu.VMEM((8, CHUNK), jnp.int32),
            pltpu.VMEM((8, CHUNK), jnp.float32),
            pltpu.VMEM((8, CHUNK), jnp.float32),
            pltpu.VMEM((8, CHUNK), jnp.float32),
            pltpu.VMEM((8, 128), jnp.float32),
            pltpu.SemaphoreType.DMA,
            pltpu.SemaphoreType.DMA,
            pltpu.VMEM_SHARED((NPAD, D), jnp.float32),
            pltpu.VMEM_SHARED((NPAD * 8,), jnp.float32),
        ],
        compiler_params=pltpu.CompilerParams(needs_layout_passes=False),
    )
    wnum, wden = kern(h, alsad.reshape(-1), gmax, src_pad, dst_pad, znum, zden)
    return wnum, wden.reshape(NC, NPAD, 8)


def _head_expand(v, H):
    cols = lax.broadcasted_iota(jnp.int32, (H, D), 1)
    rows = lax.broadcasted_iota(jnp.int32, (H, D), 0)
    ones = jnp.where(cols // (D // H) == rows, 1.0, 0.0).astype(jnp.float32)
    return jnp.dot(v, ones, preferred_element_type=jnp.float32)


def _gmax_block(gvals):
    row16 = jnp.broadcast_to(
        jnp.concatenate([gvals, jnp.zeros((1, 8), jnp.float32)], axis=1), (8, 16))
    r8 = lax.broadcasted_iota(jnp.int32, (8, 8), 0)
    c8 = lax.broadcasted_iota(jnp.int32, (8, 8), 1)
    diag = jnp.where(r8 == c8, jnp.broadcast_to(gvals, (8, 8)), 0.0)
    spl = jnp.dot(diag, jnp.ones((8, 16), jnp.float32),
                  preferred_element_type=jnp.float32)
    return jnp.concatenate([row16, spl, jnp.zeros((8, 96), jnp.float32)], axis=1)


def _tc_first_body(x_ref, w_ref, aa_ref, h_ref, alsad_ref, gmax_ref, acc_ref):
    i = pl.program_id(0)
    h = jnp.dot(x_ref[...], w_ref[...], preferred_element_type=jnp.float32)
    h_ref[...] = h
    al = jnp.dot(h, aa_ref[...], preferred_element_type=jnp.float32)
    alsad_ref[...] = al
    bm = jnp.max(al, axis=0, keepdims=True)

    @pl.when(i == 0)
    def _init():
        acc_ref[...] = jnp.full((1, 8), -jnp.inf, jnp.float32)

    acc_ref[...] = jnp.maximum(acc_ref[...], bm)

    @pl.when(i == NBLK - 1)
    def _fin():
        gmax_ref[...] = _gmax_block(acc_ref[...])


def _tc_first(x, w, aa):
    return pl.pallas_call(
        _tc_first_body,
        grid=(NBLK,),
        in_specs=[
            pl.BlockSpec((BLK, D), lambda i: (i, 0)),
            pl.BlockSpec((D, D), lambda i: (0, 0)),
            pl.BlockSpec((D, 8), lambda i: (0, 0)),
        ],
        out_specs=[
            pl.BlockSpec((BLK, D), lambda i: (i, 0)),
            pl.BlockSpec((BLK, 8), lambda i: (i, 0)),
            pl.BlockSpec((8, 128), lambda i: (0, 0)),
        ],
        out_shape=[
            jax.ShapeDtypeStruct((N, D), jnp.float32),
            jax.ShapeDtypeStruct((N, 8), jnp.float32),
            jax.ShapeDtypeStruct((8, 128), jnp.float32),
        ],
        scratch_shapes=[pltpu.VMEM((1, 8), jnp.float32)],
    )(x, w, aa)


def _combine(H, wnum, wden, h, alsad, gmax_in, b):
    als = alsad[:, 0:H]
    ald = alsad[:, H:2 * H]
    g = gmax_in[0:1, 0:H]
    t = als + ald
    lr = jnp.maximum(t, 0.2 * t)
    gd = ald + g
    M = jnp.maximum(gd, 0.2 * gd)
    sl = jnp.exp(lr - M)
    den = wden[0][:, 0:H] + wden[1][:, 0:H] + sl
    num = wnum[0] + wnum[1] + _head_expand(sl, H) * h
    return num / (_head_expand(den, H) + 1e-16) + b


def _tc_mid_body(H, wnum_ref, wden_ref, h_ref, alsad_ref, gmax_in_ref, b_ref,
                 w_ref, aa_ref, h2_ref, alsad2_ref, gmax2_ref, acc_ref):
    i = pl.program_id(0)
    o = _combine(H, wnum_ref[...], wden_ref[...], h_ref[...], alsad_ref[...],
                 gmax_in_ref[...], b_ref[...])
    act = jnp.where(o > 0, o, jnp.exp(jnp.minimum(o, 0.0)) - 1.0)
    h2 = jnp.dot(act, w_ref[...], preferred_element_type=jnp.float32)
    h2_ref[...] = h2
    al = jnp.dot(h2, aa_ref[...], preferred_element_type=jnp.float32)
    alsad2_ref[...] = al
    bm = jnp.max(al, axis=0, keepdims=True)

    @pl.when(i == 0)
    def _init():
        acc_ref[...] = jnp.full((1, 8), -jnp.inf, jnp.float32)

    acc_ref[...] = jnp.maximum(acc_ref[...], bm)

    @pl.when(i == NBLK - 1)
    def _fin():
        gmax2_ref[...] = _gmax_block(acc_ref[...])


def _tc_mid(H, wnum, wden, h, alsad, gmax_in, b, w, aa):
    return pl.pallas_call(
        functools.partial(_tc_mid_body, H),
        grid=(NBLK,),
        in_specs=[
            pl.BlockSpec((NC, BLK, D), lambda i: (0, i, 0)),
            pl.BlockSpec((NC, BLK, 8), lambda i: (0, i, 0)),
            pl.BlockSpec((BLK, D), lambda i: (i, 0)),
            pl.BlockSpec((BLK, 8), lambda i: (i, 0)),
            pl.BlockSpec((8, 128), lambda i: (0, 0)),
            pl.BlockSpec((1, D), lambda i: (0, 0)),
            pl.BlockSpec((D, D), lambda i: (0, 0)),
            pl.BlockSpec((D, 8), lambda i: (0, 0)),
        ],
        out_specs=[
            pl.BlockSpec((BLK, D), lambda i: (i, 0)),
            pl.BlockSpec((BLK, 8), lambda i: (i, 0)),
            pl.BlockSpec((8, 128), lambda i: (0, 0)),
        ],
        out_shape=[
            jax.ShapeDtypeStruct((N, D), jnp.float32),
            jax.ShapeDtypeStruct((N, 8), jnp.float32),
            jax.ShapeDtypeStruct((8, 128), jnp.float32),
        ],
        scratch_shapes=[pltpu.VMEM((1, 8), jnp.float32)],
    )(wnum, wden, h, alsad, gmax_in, b, w, aa)


def _tc_final_body(wnum_ref, wden_ref, h_ref, alsad_ref, gmax_in_ref, b_ref,
                   wc1a_ref, wc1b_ref, bc1_ref, wc2t_ref, out_ref,
                   sum_ref, max_ref):
    i = pl.program_id(0)
    o = _combine(1, wnum_ref[...], wden_ref[...], h_ref[...], alsad_ref[...],
                 gmax_in_ref[...], b_ref[...])

    @pl.when(i == 0)
    def _init():
        sum_ref[...] = jnp.zeros((1, D), jnp.float32)
        max_ref[...] = jnp.full((1, D), -jnp.inf, jnp.float32)

    sum_ref[...] = sum_ref[...] + jnp.sum(o, axis=0, keepdims=True)
    max_ref[...] = jnp.maximum(max_ref[...], jnp.max(o, axis=0, keepdims=True))

    @pl.when(i == NBLK - 1)
    def _fin():
        mean = sum_ref[...] / float(N)
        hc = jnp.dot(mean, wc1a_ref[...], preferred_element_type=jnp.float32)
        hc = hc + jnp.dot(max_ref[...], wc1b_ref[...],
                          preferred_element_type=jnp.float32)
        hc = jnp.maximum(hc + bc1_ref[...], 0.0)
        res = jnp.sum(hc * wc2t_ref[...], axis=1, keepdims=True)
        out_ref[...] = jnp.broadcast_to(res, (8, 128))


def _tc_final(wnum, wden, h, alsad, gmax_in, b, wc1a, wc1b, bc1, wc2t):
    return pl.pallas_call(
        _tc_final_body,
        grid=(NBLK,),
        in_specs=[
            pl.BlockSpec((NC, BLK, D), lambda i: (0, i, 0)),
            pl.BlockSpec((NC, BLK, 8), lambda i: (0, i, 0)),
            pl.BlockSpec((BLK, D), lambda i: (i, 0)),
            pl.BlockSpec((BLK, 8), lambda i: (i, 0)),
            pl.BlockSpec((8, 128), lambda i: (0, 0)),
            pl.BlockSpec((1, D), lambda i: (0, 0)),
            pl.BlockSpec((D, D), lambda i: (0, 0)),
            pl.BlockSpec((D, D), lambda i: (0, 0)),
            pl.BlockSpec((1, D), lambda i: (0, 0)),
            pl.BlockSpec((1, D), lambda i: (0, 0)),
        ],
        out_specs=pl.BlockSpec((8, 128), lambda i: (0, 0)),
        out_shape=jax.ShapeDtypeStruct((8, 128), jnp.float32),
        scratch_shapes=[
            pltpu.VMEM((1, D), jnp.float32),
            pltpu.VMEM((1, D), jnp.float32),
        ],
    )(wnum, wden, h, alsad, gmax_in, b, wc1a, wc1b, bc1, wc2t)


def _attn_mat(a_s, a_d, H):
    C = D // H
    eye = jnp.repeat(jnp.eye(H, dtype=jnp.float32), C, axis=0)
    As = eye * a_s.reshape(D)[:, None]
    Ad = eye * a_d.reshape(D)[:, None]
    pad = jnp.zeros((D, 8 - 2 * H), jnp.float32)
    return jnp.concatenate([As, Ad, pad], axis=1)


def kernel(x, edge_index, batch, w0, as0, ad0, b0, w1, as1, ad1, b1, w2, as2,
           ad2, b2, w3, as3, ad3, b3, wc1, bc1, wc2, bc2):
    e_real = edge_index.shape[1]
    epad = -(-e_real // (NW * CHUNK)) * (NW * CHUNK)
    src = jnp.pad(edge_index[0], (0, epad - e_real))
    dst = jnp.pad(edge_index[1], (0, epad - e_real))

    aa0 = _attn_mat(as0, ad0, 4)
    aa1 = _attn_mat(as1, ad1, 4)
    aa2 = _attn_mat(as2, ad2, 4)
    aa3 = _attn_mat(as3, ad3, 1)

    h, alsad, gmax = _tc_first(x, w0, aa0)
    for (H, w_next, aa_next, b_cur) in ((4, w1, aa1, b0), (4, w2, aa2, b1),
                                        (4, w3, aa3, b2)):
        wnum, wden = _sc_edge_call(H, h, alsad, gmax, src, dst, e_real)
        wnum, wden = wnum[:, :N], wden[:, :N]
        h, alsad, gmax = _tc_mid(H, wnum, wden, h, alsad, gmax,
                                 b_cur.reshape(1, D), w_next, aa_next)
    wnum, wden = _sc_edge_call(1, h, alsad, gmax, src, dst, e_real)
    wnum, wden = wnum[:, :N], wden[:, :N]
    out = _tc_final(wnum, wden, h, alsad, gmax, b3.reshape(1, D),
                    wc1[:D], wc1[D:], bc1.reshape(1, D), wc2.reshape(1, D))
    return out[0:1, 0:1]


# speedup vs baseline: 3.6650x; 3.6650x over previous
import functools

import jax
import jax.numpy as jnp
from jax import lax
from jax.experimental import pallas as pl
from jax.experimental.pallas import tpu as pltpu
from jax.experimental.pallas import tpu_sc as plsc

N = 10000
D = 128
NC = 2
NS = 16
NW = NC * NS
NPAD = 10240
ROWS_PER_TILE = NPAD // NS
CHUNK = 128
NBLK = 10
BLK = N // NBLK


def _sc_edge_body(H, E_real, h_hbm, alsad_hbm, gmax_hbm, src_hbm, dst_hbm,
                  znum_hbm, zden_hbm, wnum_out, wden_out, src_v, dst_v, rows_v,
                  aidx_v, adidx_v, didx_v, as_vals, ad_vals, svals_v, gmax_v,
                  sem, sem2, wnum_s, wden_s):
    C = D // H
    cid = lax.axis_index("c")
    sid = lax.axis_index("s")
    wid = sid * NC + cid
    EPAD = src_hbm.shape[0]
    per_tile = EPAD // NW
    n_chunks = per_tile // CHUNK

    pltpu.sync_copy(gmax_hbm, gmax_v)

    pltpu.sync_copy(znum_hbm, wnum_s.at[pl.ds(sid * ROWS_PER_TILE, ROWS_PER_TILE)])
    pltpu.sync_copy(zden_hbm, wden_s.at[pl.ds(sid * ROWS_PER_TILE * 8,
                                              ROWS_PER_TILE * 8)])
    plsc.subcore_barrier()

    lanes = lax.broadcasted_iota(jnp.int32, (16,), 0)

    def chunk_body(ch, carry):
        base = wid * per_tile + ch * CHUNK
        di = pltpu.async_copy(src_hbm.at[pl.ds(base, CHUNK)], src_v, sem)
        dj = pltpu.async_copy(dst_hbm.at[pl.ds(base, CHUNK)], dst_v, sem)
        di.wait()
        dj.wait()
        descs = [pltpu.async_copy(h_hbm.at[src_v], rows_v, sem)]
        for j in range(CHUNK // 16):
            jsl = pl.ds(j * 16, 16)
            sv = src_v[jsl]
            dv = dst_v[jsl]
            for hd in range(H):
                aidx_v[hd, jsl] = sv * 8 + hd
                adidx_v[hd, jsl] = dv * 8 + (H + hd)
                didx_v[hd, jsl] = dv * 8 + hd
        for hd in range(H):
            descs.append(pltpu.async_copy(
                alsad_hbm.at[aidx_v.at[hd]], as_vals.at[hd], sem2))
            descs.append(pltpu.async_copy(
                alsad_hbm.at[adidx_v.at[hd]], ad_vals.at[hd], sem2))
        for de in descs:
            de.wait()
        for j in range(CHUNK // 16):
            j16 = j * 16 + lanes
            jsl = pl.ds(j * 16, 16)
            valid = (base + j16) < E_real
            svecs = []
            for hd in range(H):
                ts = as_vals[hd, jsl]
                td = ad_vals[hd, jsl]
                a = ts + td
                lr = jnp.maximum(a, 0.2 * a)
                gs = gmax_v[hd, pl.ds(16, 16)]
                gd = td + gs
                M = jnp.maximum(gd, 0.2 * gd)
                s = jnp.where(valid, jnp.exp(lr - M), 0.0)
                svals_v[hd, jsl] = s
                svecs.append(s)
            for e16 in range(16):
                e = j * 16 + e16
                esel = jnp.full((16,), e16, jnp.int32)
                for hd in range(H):
                    spl = jnp.take(svecs[hd], esel)
                    for c2 in range(C // 16):
                        csl = pl.ds(hd * C + c2 * 16, 16)
                        rows_v[e, csl] = rows_v[e, csl] * spl
        outs = [pltpu.async_copy(rows_v, wnum_s.at[dst_v], sem, add=True)]
        for hd in range(H):
            outs.append(pltpu.async_copy(
                svals_v.at[hd], wden_s.at[didx_v.at[hd]], sem2, add=True))
        for de in outs:
            de.wait()
        return carry

    lax.fori_loop(0, n_chunks, chunk_body, 0)
    plsc.subcore_barrier()

    rsl = pl.ds(sid * ROWS_PER_TILE, ROWS_PER_TILE)
    fsl = pl.ds(sid * ROWS_PER_TILE * 8, ROWS_PER_TILE * 8)
    pltpu.sync_copy(wnum_s.at[rsl], wnum_out.at[cid, rsl])
    pltpu.sync_copy(wden_s.at[fsl], wden_out.at[cid, fsl])


def _sc_edge_call(H, h, alsad, gmax, src_pad, dst_pad, e_real):
    mesh = plsc.VectorSubcoreMesh(core_axis_name="c", subcore_axis_name="s")
    znum = jnp.zeros((ROWS_PER_TILE, D), jnp.float32)
    zden = jnp.zeros((ROWS_PER_TILE * 8,), jnp.float32)
    kern = pl.kernel(
        functools.partial(_sc_edge_body, H, e_real),
        out_type=(
            jax.ShapeDtypeStruct((NC, NPAD, D), jnp.float32),
            jax.ShapeDtypeStruct((NC, NPAD * 8), jnp.float32),
        ),
        mesh=mesh,
        scratch_types=[
            pltpu.VMEM((CHUNK,), jnp.int32),
            pltpu.VMEM((CHUNK,), jnp.int32),
            pltpu.VMEM((CHUNK, D), jnp.float32),
            pltpu.VMEM((8, CHUNK), jnp.int32),
            pltpu.VMEM((8, CHUNK), jnp.int32),
            pltpu.VMEM((8, CHUNK), jnp.int32),
            pltpu.VMEM((8, CHUNK), jnp.float32),
            pltpu.VMEM((8, CHUNK), jnp.float32),
            pltpu.VMEM((8, CHUNK), jnp.float32),
            pltpu.VMEM((8, 128), jnp.float32),
            pltpu.SemaphoreType.DMA,
            pltpu.SemaphoreType.DMA,
            pltpu.VMEM_SHARED((NPAD, D), jnp.float32),
            pltpu.VMEM_SHARED((NPAD * 8,), jnp.float32),
        ],
        compiler_params=pltpu.CompilerParams(needs_layout_passes=False),
    )
    wnum, wden = kern(h, alsad.reshape(-1), gmax, src_pad, dst_pad, znum, zden)
    return wnum, wden.reshape(NC, NPAD, 8)


def _head_expand(v, H):
    cols = lax.broadcasted_iota(jnp.int32, (H, D), 1)
    rows = lax.broadcasted_iota(jnp.int32, (H, D), 0)
    ones = jnp.where(cols // (D // H) == rows, 1.0, 0.0).astype(jnp.float32)
    return jnp.dot(v, ones, preferred_element_type=jnp.float32)


def _gmax_block(gvals):
    row16 = jnp.broadcast_to(
        jnp.concatenate([gvals, jnp.zeros((1, 8), jnp.float32)], axis=1), (8, 16))
    r8 = lax.broadcasted_iota(jnp.int32, (8, 8), 0)
    c8 = lax.broadcasted_iota(jnp.int32, (8, 8), 1)
    diag = jnp.where(r8 == c8, jnp.broadcast_to(gvals, (8, 8)), 0.0)
    spl = jnp.dot(diag, jnp.ones((8, 16), jnp.float32),
                  preferred_element_type=jnp.float32)
    return jnp.concatenate([row16, spl, jnp.zeros((8, 96), jnp.float32)], axis=1)


def _tc_first_body(x_ref, w_ref, aa_ref, h_ref, alsad_ref, gmax_ref, acc_ref):
    i = pl.program_id(0)
    h = jnp.dot(x_ref[...], w_ref[...], preferred_element_type=jnp.float32)
    h_ref[...] = h
    al = jnp.dot(h, aa_ref[...], preferred_element_type=jnp.float32)
    alsad_ref[...] = al
    bm = jnp.max(al, axis=0, keepdims=True)

    @pl.when(i == 0)
    def _init():
        acc_ref[...] = jnp.full((1, 8), -jnp.inf, jnp.float32)

    acc_ref[...] = jnp.maximum(acc_ref[...], bm)

    @pl.when(i == NBLK - 1)
    def _fin():
        gmax_ref[...] = _gmax_block(acc_ref[...])


def _tc_first(x, w, aa):
    return pl.pallas_call(
        _tc_first_body,
        grid=(NBLK,),
        in_specs=[
            pl.BlockSpec((BLK, D), lambda i: (i, 0)),
            pl.BlockSpec((D, D), lambda i: (0, 0)),
            pl.BlockSpec((D, 8), lambda i: (0, 0)),
        ],
        out_specs=[
            pl.BlockSpec((BLK, D), lambda i: (i, 0)),
            pl.BlockSpec((BLK, 8), lambda i: (i, 0)),
            pl.BlockSpec((8, 128), lambda i: (0, 0)),
        ],
        out_shape=[
            jax.ShapeDtypeStruct((N, D), jnp.float32),
            jax.ShapeDtypeStruct((N, 8), jnp.float32),
            jax.ShapeDtypeStruct((8, 128), jnp.float32),
        ],
        scratch_shapes=[pltpu.VMEM((1, 8), jnp.float32)],
    )(x, w, aa)


def _combine(H, wnum, wden, h, alsad, gmax_in, b):
    als = alsad[:, 0:H]
    ald = alsad[:, H:2 * H]
    g = gmax_in[0:1, 0:H]
    t = als + ald
    lr = jnp.maximum(t, 0.2 * t)
    gd = ald + g
    M = jnp.maximum(gd, 0.2 * gd)
    sl = jnp.exp(lr - M)
    den = wden[0][:, 0:H] + wden[1][:, 0:H] + sl
    num = wnum[0] + wnum[1] + _head_expand(sl, H) * h
    return num / (_head_expand(den, H) + 1e-16) + b


def _tc_mid_body(H, wnum_ref, wden_ref, h_ref, alsad_ref, gmax_in_ref, b_ref,
                 w_ref, aa_ref, h2_ref, alsad2_ref, gmax2_ref, acc_ref):
    i = pl.program_id(0)
    o = _combine(H, wnum_ref[...], wden_ref[...], h_ref[...], alsad_ref[...],
                 gmax_in_ref[...], b_ref[...])
    act = jnp.where(o > 0, o, jnp.exp(jnp.minimum(o, 0.0)) - 1.0)
    h2 = jnp.dot(act, w_ref[...], preferred_element_type=jnp.float32)
    h2_ref[...] = h2
    al = jnp.dot(h2, aa_ref[...], preferred_element_type=jnp.float32)
    alsad2_ref[...] = al
    bm = jnp.max(al, axis=0, keepdims=True)

    @pl.when(i == 0)
    def _init():
        acc_ref[...] = jnp.full((1, 8), -jnp.inf, jnp.float32)

    acc_ref[...] = jnp.maximum(acc_ref[...], bm)

    @pl.when(i == NBLK - 1)
    def _fin():
        gmax2_ref[...] = _gmax_block(acc_ref[...])


def _tc_mid(H, wnum, wden, h, alsad, gmax_in, b, w, aa):
    return pl.pallas_call(
        functools.partial(_tc_mid_body, H),
        grid=(NBLK,),
        in_specs=[
            pl.BlockSpec((NC, BLK, D), lambda i: (0, i, 0)),
            pl.BlockSpec((NC, BLK, 8), lambda i: (0, i, 0)),
            pl.BlockSpec((BLK, D), lambda i: (i, 0)),
            pl.BlockSpec((BLK, 8), lambda i: (i, 0)),
            pl.BlockSpec((8, 128), lambda i: (0, 0)),
            pl.BlockSpec((1, D), lambda i: (0, 0)),
            pl.BlockSpec((D, D), lambda i: (0, 0)),
            pl.BlockSpec((D, 8), lambda i: (0, 0)),
        ],
        out_specs=[
            pl.BlockSpec((BLK, D), lambda i: (i, 0)),
            pl.BlockSpec((BLK, 8), lambda i: (i, 0)),
            pl.BlockSpec((8, 128), lambda i: (0, 0)),
        ],
        out_shape=[
            jax.ShapeDtypeStruct((N, D), jnp.float32),
            jax.ShapeDtypeStruct((N, 8), jnp.float32),
            jax.ShapeDtypeStruct((8, 128), jnp.float32),
        ],
        scratch_shapes=[pltpu.VMEM((1, 8), jnp.float32)],
    )(wnum, wden, h, alsad, gmax_in, b, w, aa)


def _tc_final_body(wnum_ref, wden_ref, h_ref, alsad_ref, gmax_in_ref, b_ref,
                   wc1a_ref, wc1b_ref, bc1_ref, wc2t_ref, out_ref,
                   sum_ref, max_ref):
    i = pl.program_id(0)
    o = _combine(1, wnum_ref[...], wden_ref[...], h_ref[...], alsad_ref[...],
                 gmax_in_ref[...], b_ref[...])

    @pl.when(i == 0)
    def _init():
        sum_ref[...] = jnp.zeros((1, D), jnp.float32)
        max_ref[...] = jnp.full((1, D), -jnp.inf, jnp.float32)

    sum_ref[...] = sum_ref[...] + jnp.sum(o, axis=0, keepdims=True)
    max_ref[...] = jnp.maximum(max_ref[...], jnp.max(o, axis=0, keepdims=True))

    @pl.when(i == NBLK - 1)
    def _fin():
        mean = sum_ref[...] / float(N)
        hc = jnp.dot(mean, wc1a_ref[...], preferred_element_type=jnp.float32)
        hc = hc + jnp.dot(max_ref[...], wc1b_ref[...],
                          preferred_element_type=jnp.float32)
        hc = jnp.maximum(hc + bc1_ref[...], 0.0)
        res = jnp.sum(hc * wc2t_ref[...], axis=1, keepdims=True)
        out_ref[...] = jnp.broadcast_to(res, (8, 128))


def _tc_final(wnum, wden, h, alsad, gmax_in, b, wc1a, wc1b, bc1, wc2t):
    return pl.pallas_call(
        _tc_final_body,
        grid=(NBLK,),
        in_specs=[
            pl.BlockSpec((NC, BLK, D), lambda i: (0, i, 0)),
            pl.BlockSpec((NC, BLK, 8), lambda i: (0, i, 0)),
            pl.BlockSpec((BLK, D), lambda i: (i, 0)),
            pl.BlockSpec((BLK, 8), lambda i: (i, 0)),
            pl.BlockSpec((8, 128), lambda i: (0, 0)),
            pl.BlockSpec((1, D), lambda i: (0, 0)),
            pl.BlockSpec((D, D), lambda i: (0, 0)),
            pl.BlockSpec((D, D), lambda i: (0, 0)),
            pl.BlockSpec((1, D), lambda i: (0, 0)),
            pl.BlockSpec((1, D), lambda i: (0, 0)),
        ],
        out_specs=pl.BlockSpec((8, 128), lambda i: (0, 0)),
        out_shape=jax.ShapeDtypeStruct((8, 128), jnp.float32),
        scratch_shapes=[
            pltpu.VMEM((1, D), jnp.float32),
            pltpu.VMEM((1, D), jnp.float32),
        ],
    )(wnum, wden, h, alsad, gmax_in, b, wc1a, wc1b, bc1, wc2t)


def _attn_mat(a_s, a_d, H):
    C = D // H
    eye = jnp.repeat(jnp.eye(H, dtype=jnp.float32), C, axis=0)
    As = eye * a_s.reshape(D)[:, None]
    Ad = eye * a_d.reshape(D)[:, None]
    pad = jnp.zeros((D, 8 - 2 * H), jnp.float32)
    return jnp.concatenate([As, Ad, pad], axis=1)


def kernel(x, edge_index, batch, w0, as0, ad0, b0, w1, as1, ad1, b1, w2, as2,
           ad2, b2, w3, as3, ad3, b3, wc1, bc1, wc2, bc2):
    e_real = edge_index.shape[1]
    epad = -(-e_real // (NW * CHUNK)) * (NW * CHUNK)
    src = jnp.pad(edge_index[0], (0, epad - e_real))
    dst = jnp.pad(edge_index[1], (0, epad - e_real))

    aa0 = _attn_mat(as0, ad0, 4)
    aa1 = _attn_mat(as1, ad1, 4)
    aa2 = _attn_mat(as2, ad2, 4)
    aa3 = _attn_mat(as3, ad3, 1)

    h, alsad, gmax = _tc_first(x, w0, aa0)
    for (H, w_next, aa_next, b_cur) in ((4, w1, aa1, b0), (4, w2, aa2, b1),
                                        (4, w3, aa3, b2)):
        wnum, wden = _sc_edge_call(H, h, alsad, gmax, src, dst, e_real)
        wnum, wden = wnum[:, :N], wden[:, :N]
        h, alsad, gmax = _tc_mid(H, wnum, wden, h, alsad, gmax,
                                 b_cur.reshape(1, D), w_next, aa_next)
    wnum, wden = _sc_edge_call(1, h, alsad, gmax, src, dst, e_real)
    wnum, wden = wnum[:, :N], wden[:, :N]
    out = _tc_final(wnum, wden, h, alsad, gmax, b3.reshape(1, D),
                    wc1[:D], wc1[D:], bc1.reshape(1, D), wc2.reshape(1, D))
    return out[0:1, 0:1]
